# baseline (speedup 1.0000x reference)
import jax
import jax.numpy as jnp
from jax.experimental import pallas as pl
from jax.experimental.pallas import tpu as pltpu

_TM = 256
_TN = 512


def _linear_kernel(x_ref, w_ref, b_ref, o_ref, wbf_ref, xbf_ref):
    i = pl.program_id(0)
    j = pl.program_id(1)

    @pl.when(i == 0)
    def _():
        wbf_ref[:, pl.ds(j * _TN, _TN)] = w_ref[...].astype(jnp.bfloat16)

    @pl.when(j == 0)
    def _():
        xbf_ref[...] = x_ref[...].astype(jnp.bfloat16)

    o_ref[...] = (
        jnp.dot(
            xbf_ref[...],
            wbf_ref[:, pl.ds(j * _TN, _TN)],
            preferred_element_type=jnp.float32,
        )
        + b_ref[...]
    ).astype(o_ref.dtype)


@jax.jit
def _linear_call(x, w_t, b2):
    M, K = x.shape
    _, N = w_t.shape
    grid = (pl.cdiv(M, _TM), pl.cdiv(N, _TN))

    def _w_index(i, j):
        return (0, jnp.where(i == 0, j, 0))

    return pl.pallas_call(
        _linear_kernel,
        out_shape=jax.ShapeDtypeStruct((M, N), jnp.float32),
        grid_spec=pltpu.PrefetchScalarGridSpec(
            num_scalar_prefetch=0,
            grid=grid,
            in_specs=[
                pl.BlockSpec((_TM, K), lambda i, j: (i, 0)),
                pl.BlockSpec((K, _TN), _w_index),
                pl.BlockSpec((1, _TN), lambda i, j: (0, j)),
            ],
            out_specs=pl.BlockSpec((_TM, _TN), lambda i, j: (i, j)),
            scratch_shapes=[
                pltpu.VMEM((K, N), jnp.bfloat16),
                pltpu.VMEM((_TM, K), jnp.bfloat16),
            ],
        ),
        compiler_params=pltpu.CompilerParams(
            dimension_semantics=("arbitrary", "arbitrary"),
            vmem_limit_bytes=62 * 1024 * 1024,
        ),
    )(x, w_t, b2)


def kernel(x, w_t, b2):
    return _linear_call(x, w_t, b2)


# speedup vs baseline: 1.3071x; 1.3071x over previous
import functools

import jax
import jax.numpy as jnp
from jax.experimental import pallas as pl
from jax.experimental.pallas import tpu as pltpu


def _linear_kernel(x_ref, w_ref, b_ref, o_ref):
    x = x_ref[...].astype(jnp.bfloat16)
    o_ref[...] = (
        jnp.dot(x, w_ref[...], preferred_element_type=jnp.float32)
        + b_ref[...]
    ).astype(o_ref.dtype)


@functools.partial(jax.jit, static_argnames=("tm",))
def _linear_call(x, w_bf, b2, tm):
    M, K = x.shape
    _, N = w_bf.shape
    grid = (pl.cdiv(M, tm),)
    return pl.pallas_call(
        _linear_kernel,
        out_shape=jax.ShapeDtypeStruct((M, N), jnp.float32),
        grid_spec=pltpu.PrefetchScalarGridSpec(
            num_scalar_prefetch=0,
            grid=grid,
            in_specs=[
                pl.BlockSpec((tm, K), lambda i: (i, 0)),
                pl.BlockSpec((K, N), lambda i: (0, 0)),
                pl.BlockSpec((1, N), lambda i: (0, 0)),
            ],
            out_specs=pl.BlockSpec((tm, N), lambda i: (i, 0)),
        ),
        compiler_params=pltpu.CompilerParams(
            dimension_semantics=("parallel",),
            vmem_limit_bytes=60 * 1024 * 1024,
        ),
    )(x, w_bf, b2)


def kernel(x, w_t, b2):
    w_bf = w_t.astype(jnp.bfloat16)
    return _linear_call(x, w_bf, b2, tm=256)


# speedup vs baseline: 1.3658x; 1.0449x over previous
import jax
import jax.numpy as jnp
from jax.experimental import pallas as pl
from jax.experimental.pallas import tpu as pltpu

_TM_A = 1024
_TC = 256
_TM_B = 256


def _cast_row_kernel(x_ref, w_ref, b_ref, y_ref, wbf_ref, xbf_ref):
    @pl.when(pl.program_id(0) == 0)
    def _():
        xbf_ref[...] = x_ref[...].astype(jnp.bfloat16)

    wc = w_ref[...].astype(jnp.bfloat16)
    wbf_ref[...] = wc
    y_ref[...] = (
        jnp.dot(xbf_ref[...], wc, preferred_element_type=jnp.float32)
        + b_ref[...]
    ).astype(y_ref.dtype)


def _main_kernel(x_ref, w_ref, b_ref, y_in_ref, o_ref):
    del y_in_ref
    x = x_ref[...].astype(jnp.bfloat16)
    o_ref[...] = (
        jnp.dot(x, w_ref[...], preferred_element_type=jnp.float32)
        + b_ref[...]
    ).astype(o_ref.dtype)


@jax.jit
def _linear_call(x, w_t, b2):
    M, K = x.shape
    _, N = w_t.shape

    y_top, w_bf = pl.pallas_call(
        _cast_row_kernel,
        out_shape=(
            jax.ShapeDtypeStruct((M, N), jnp.float32),
            jax.ShapeDtypeStruct((K, N), jnp.bfloat16),
        ),
        grid_spec=pltpu.PrefetchScalarGridSpec(
            num_scalar_prefetch=0,
            grid=(N // _TC,),
            in_specs=[
                pl.BlockSpec((_TM_A, K), lambda j: (0, 0)),
                pl.BlockSpec((K, _TC), lambda j: (0, j)),
                pl.BlockSpec((1, _TC), lambda j: (0, j)),
            ],
            out_specs=(
                pl.BlockSpec((_TM_A, _TC), lambda j: (0, j)),
                pl.BlockSpec((K, _TC), lambda j: (0, j)),
            ),
            scratch_shapes=[
                pltpu.VMEM((_TM_A, K), jnp.bfloat16),
            ],
        ),
        compiler_params=pltpu.CompilerParams(
            dimension_semantics=("arbitrary",),
            vmem_limit_bytes=60 * 1024 * 1024,
        ),
    )(x, w_t, b2)

    row0 = _TM_A // _TM_B
    return pl.pallas_call(
        _main_kernel,
        out_shape=jax.ShapeDtypeStruct((M, N), jnp.float32),
        grid_spec=pltpu.PrefetchScalarGridSpec(
            num_scalar_prefetch=0,
            grid=((M - _TM_A) // _TM_B,),
            in_specs=[
                pl.BlockSpec((_TM_B, K), lambda i: (i + row0, 0)),
                pl.BlockSpec((K, N), lambda i: (0, 0)),
                pl.BlockSpec((1, N), lambda i: (0, 0)),
                pl.BlockSpec(memory_space=pl.ANY),
            ],
            out_specs=pl.BlockSpec((_TM_B, N), lambda i: (i + row0, 0)),
        ),
        input_output_aliases={3: 0},
        compiler_params=pltpu.CompilerParams(
            dimension_semantics=("parallel",),
            vmem_limit_bytes=60 * 1024 * 1024,
        ),
    )(x, w_bf, b2, y_top)


def kernel(x, w_t, b2):
    return _linear_call(x, w_t, b2)


# speedup vs baseline: 1.3768x; 1.0080x over previous
import jax
import jax.numpy as jnp
from jax.experimental import pallas as pl
from jax.experimental.pallas import tpu as pltpu

_TM_A = 1024
_TC = 512
_TM_B = 256


def _cast_row_kernel(x_ref, w_ref, b_ref, y_ref, wbf_ref, xbf_ref):
    @pl.when(pl.program_id(0) == 0)
    def _():
        xbf_ref[...] = x_ref[...].astype(jnp.bfloat16)

    wc = w_ref[...].astype(jnp.bfloat16)
    wbf_ref[...] = wc
    y_ref[...] = (
        jnp.dot(xbf_ref[...], wc, preferred_element_type=jnp.float32)
        + b_ref[...]
    ).astype(y_ref.dtype)


def _main_kernel(x_ref, w_ref, b_ref, y_in_ref, o_ref):
    del y_in_ref
    x = x_ref[...].astype(jnp.bfloat16)
    o_ref[...] = (
        jnp.dot(x, w_ref[...], preferred_element_type=jnp.float32)
        + b_ref[...]
    ).astype(o_ref.dtype)


@jax.jit
def _linear_call(x, w_t, b2):
    M, K = x.shape
    _, N = w_t.shape

    y_top, w_bf = pl.pallas_call(
        _cast_row_kernel,
        out_shape=(
            jax.ShapeDtypeStruct((M, N), jnp.float32),
            jax.ShapeDtypeStruct((K, N), jnp.bfloat16),
        ),
        grid_spec=pltpu.PrefetchScalarGridSpec(
            num_scalar_prefetch=0,
            grid=(N // _TC,),
            in_specs=[
                pl.BlockSpec((_TM_A, K), lambda j: (0, 0)),
                pl.BlockSpec((K, _TC), lambda j: (0, j)),
                pl.BlockSpec((1, _TC), lambda j: (0, j)),
            ],
            out_specs=(
                pl.BlockSpec((_TM_A, _TC), lambda j: (0, j)),
                pl.BlockSpec((K, _TC), lambda j: (0, j)),
            ),
            scratch_shapes=[
                pltpu.VMEM((_TM_A, K), jnp.bfloat16),
            ],
        ),
        compiler_params=pltpu.CompilerParams(
            dimension_semantics=("arbitrary",),
            vmem_limit_bytes=60 * 1024 * 1024,
        ),
    )(x, w_t, b2)

    row0 = _TM_A // _TM_B
    return pl.pallas_call(
        _main_kernel,
        out_shape=jax.ShapeDtypeStruct((M, N), jnp.float32),
        grid_spec=pltpu.PrefetchScalarGridSpec(
            num_scalar_prefetch=0,
            grid=((M - _TM_A) // _TM_B,),
            in_specs=[
                pl.BlockSpec((_TM_B, K), lambda i: (i + row0, 0)),
                pl.BlockSpec((K, N), lambda i: (0, 0)),
                pl.BlockSpec((1, N), lambda i: (0, 0)),
                pl.BlockSpec(memory_space=pl.ANY),
            ],
            out_specs=pl.BlockSpec((_TM_B, N), lambda i: (i + row0, 0)),
        ),
        input_output_aliases={3: 0},
        compiler_params=pltpu.CompilerParams(
            dimension_semantics=("parallel",),
            vmem_limit_bytes=60 * 1024 * 1024,
        ),
    )(x, w_bf, b2, y_top)


def kernel(x, w_t, b2):
    return _linear_call(x, w_t, b2)
